# Initial kernel scaffold; baseline (speedup 1.0000x reference)
import jax
import jax.numpy as jnp
from jax.experimental import pallas as pl


def kernel(v, e, edge_index, W1, b1, W2, b2):
    raise NotImplementedError("write your pallas kernel here")


# speedup vs baseline: 4.3090x; 4.3090x over previous
import functools

import jax
import jax.numpy as jnp
from jax import lax
from jax.experimental import pallas as pl
from jax.experimental.pallas import tpu as pltpu
from jax.experimental.pallas import tpu_sc as plsc

N_NODES = 10000
N_EDGES = 320000
HID = 128

NC = 2
NS = 16
NW = NC * NS
GROUP = 80
G_PER_W = N_EDGES // (NW * GROUP)
N_PAD = 10240
ROWS_PER_SUB = N_PAD // NS
ZROWS = 80


def _sc_segment_sum(e, idx3d):
    mesh = plsc.VectorSubcoreMesh(core_axis_name="c", subcore_axis_name="s")

    @functools.partial(
        pl.kernel,
        mesh=mesh,
        out_type=jax.ShapeDtypeStruct((NC, N_PAD, HID), jnp.float32),
        scratch_types=[
            pltpu.VMEM((G_PER_W, GROUP), jnp.int32),
            pltpu.VMEM((GROUP, HID), jnp.float32),
            pltpu.VMEM((ZROWS, HID), jnp.float32),
            pltpu.VMEM_SHARED((N_PAD, HID), jnp.float32),
        ],
    )
    def k(e_hbm, idx_hbm, out_hbm, idx_v, rows_v, zero_v, acc):
        c = lax.axis_index("c")
        s = lax.axis_index("s")
        wid = c * NS + s

        def zbody(t, carry):
            r = t // (HID // 16)
            col = t % (HID // 16)
            zero_v[r, pl.ds(col * 16, 16)] = jnp.zeros((16,), jnp.float32)
            return carry

        lax.fori_loop(0, ZROWS * (HID // 16), zbody, 0)

        base_row = s * ROWS_PER_SUB
        for t in range(ROWS_PER_SUB // ZROWS):
            pltpu.sync_copy(zero_v, acc.at[pl.ds(base_row + t * ZROWS, ZROWS)])
        plsc.subcore_barrier()

        pltpu.sync_copy(idx_hbm.at[wid], idx_v)

        def body(g, carry):
            eb = (wid * G_PER_W + g) * GROUP
            pltpu.sync_copy(e_hbm.at[pl.ds(eb, GROUP)], rows_v)
            pltpu.sync_copy(rows_v, acc.at[idx_v.at[g]], add=True)
            return carry

        lax.fori_loop(0, G_PER_W, body, 0)
        plsc.subcore_barrier()

        pltpu.sync_copy(
            acc.at[pl.ds(base_row, ROWS_PER_SUB)],
            out_hbm.at[c, pl.ds(base_row, ROWS_PER_SUB)],
        )

    return k(e, idx3d)


_LOG2 = 0.6931471805599453
_BLK = 1000


def _mlp_body(p0_ref, p1_ref, v_ref, w1t_ref, b1_ref, w2t_ref, b2_ref, o_ref):
    x = p0_ref[...] + p1_ref[...]
    h = jnp.dot(x, w1t_ref[...], preferred_element_type=jnp.float32) + b1_ref[...]
    h = jnp.maximum(h, 0.0) + jnp.log1p(jnp.exp(-jnp.abs(h))) - _LOG2
    y = jnp.dot(h, w2t_ref[...], preferred_element_type=jnp.float32) + b2_ref[...]
    o_ref[...] = v_ref[...] + y


def _tc_mlp(p0, p1, v, w1t, b1, w2t, b2):
    n = v.shape[0]
    grid = (n // _BLK,)
    row_spec = pl.BlockSpec((_BLK, HID), lambda i: (i, 0))
    full_spec = pl.BlockSpec((HID, HID), lambda i: (0, 0))
    bias_spec = pl.BlockSpec((1, HID), lambda i: (0, 0))
    return pl.pallas_call(
        _mlp_body,
        grid=grid,
        in_specs=[row_spec, row_spec, row_spec, full_spec, bias_spec,
                  full_spec, bias_spec],
        out_specs=row_spec,
        out_shape=jax.ShapeDtypeStruct((n, HID), jnp.float32),
    )(p0, p1, v, w1t, b1, w2t, b2)


def kernel(v, e, edge_index, W1, b1, W2, b2):
    idx3d = edge_index[1].reshape(NW, G_PER_W, GROUP)
    partial = _sc_segment_sum(e, idx3d)
    return _tc_mlp(
        partial[0, :N_NODES], partial[1, :N_NODES], v,
        W1.T, b1.reshape(1, HID), W2.T, b2.reshape(1, HID),
    )


# speedup vs baseline: 6.5667x; 1.5239x over previous
import functools

import jax
import jax.numpy as jnp
from jax import lax
from jax.experimental import pallas as pl
from jax.experimental.pallas import tpu as pltpu
from jax.experimental.pallas import tpu_sc as plsc

N_NODES = 10000
N_EDGES = 320000
HID = 128

NC = 2
NS = 16
NW = NC * NS
GROUP = 80
G_PER_W = N_EDGES // (NW * GROUP)
N_PAD = 10240
ROWS_PER_SUB = N_PAD // NS
ZROWS = 80


def _sc_segment_sum(e, idx3d):
    mesh = plsc.VectorSubcoreMesh(core_axis_name="c", subcore_axis_name="s")

    @functools.partial(
        pl.kernel,
        mesh=mesh,
        out_type=jax.ShapeDtypeStruct((NC, N_PAD, HID), jnp.float32),
        scratch_types=[
            pltpu.VMEM((G_PER_W, GROUP), jnp.int32),
            pltpu.VMEM((GROUP, HID), jnp.float32),
            pltpu.VMEM((GROUP, HID), jnp.float32),
            pltpu.VMEM((ZROWS, HID), jnp.float32),
            pltpu.VMEM_SHARED((N_PAD, HID), jnp.float32),
            pltpu.SemaphoreType.DMA,
            pltpu.SemaphoreType.DMA,
        ],
    )
    def k(e_hbm, idx_hbm, out_hbm, idx_v, rows0_v, rows1_v, zero_v, acc,
          sem0, sem1):
        c = lax.axis_index("c")
        s = lax.axis_index("s")
        wid = c * NS + s

        def zbody(t, carry):
            r = t // (HID // 16)
            col = t % (HID // 16)
            zero_v[r, pl.ds(col * 16, 16)] = jnp.zeros((16,), jnp.float32)
            return carry

        lax.fori_loop(0, ZROWS * (HID // 16), zbody, 0)

        base_row = s * ROWS_PER_SUB
        for t in range(ROWS_PER_SUB // ZROWS):
            pltpu.sync_copy(zero_v, acc.at[pl.ds(base_row + t * ZROWS, ZROWS)])
        plsc.subcore_barrier()

        pltpu.sync_copy(idx_hbm.at[wid], idx_v)

        ebase = wid * G_PER_W * GROUP

        def load(g, buf, sem):
            pltpu.async_copy(e_hbm.at[pl.ds(ebase + g * GROUP, GROUP)],
                             buf, sem)

        def drain(buf, sem):
            pltpu.make_async_copy(e_hbm.at[pl.ds(0, GROUP)], buf, sem).wait()

        load(0, rows0_v, sem0)

        def body(k2, carry):
            k = k2 * 2
            load(k + 1, rows1_v, sem1)
            drain(rows0_v, sem0)
            pltpu.sync_copy(rows0_v, acc.at[idx_v.at[k]], add=True)
            load(k + 2, rows0_v, sem0)
            drain(rows1_v, sem1)
            pltpu.sync_copy(rows1_v, acc.at[idx_v.at[k + 1]], add=True)
            return carry

        lax.fori_loop(0, (G_PER_W - 1) // 2, body, 0)
        drain(rows0_v, sem0)
        pltpu.sync_copy(rows0_v, acc.at[idx_v.at[G_PER_W - 1]], add=True)
        plsc.subcore_barrier()

        pltpu.sync_copy(
            acc.at[pl.ds(base_row, ROWS_PER_SUB)],
            out_hbm.at[c, pl.ds(base_row, ROWS_PER_SUB)],
        )

    return k(e, idx3d)


_LOG2 = 0.6931471805599453
_BLK = 1000


def _mlp_body(p0_ref, p1_ref, v_ref, w1t_ref, b1_ref, w2t_ref, b2_ref, o_ref):
    x = p0_ref[...] + p1_ref[...]
    h = jnp.dot(x, w1t_ref[...], preferred_element_type=jnp.float32) + b1_ref[...]
    h = jnp.maximum(h, 0.0) + jnp.log1p(jnp.exp(-jnp.abs(h))) - _LOG2
    y = jnp.dot(h, w2t_ref[...], preferred_element_type=jnp.float32) + b2_ref[...]
    o_ref[...] = v_ref[...] + y


def _tc_mlp(p0, p1, v, w1t, b1, w2t, b2):
    n = v.shape[0]
    grid = (n // _BLK,)
    row_spec = pl.BlockSpec((_BLK, HID), lambda i: (i, 0))
    full_spec = pl.BlockSpec((HID, HID), lambda i: (0, 0))
    bias_spec = pl.BlockSpec((1, HID), lambda i: (0, 0))
    return pl.pallas_call(
        _mlp_body,
        grid=grid,
        in_specs=[row_spec, row_spec, row_spec, full_spec, bias_spec,
                  full_spec, bias_spec],
        out_specs=row_spec,
        out_shape=jax.ShapeDtypeStruct((n, HID), jnp.float32),
    )(p0, p1, v, w1t, b1, w2t, b2)


def kernel(v, e, edge_index, W1, b1, W2, b2):
    idx3d = edge_index[1].reshape(NW, G_PER_W, GROUP)
    partial = _sc_segment_sum(e, idx3d)
    return _tc_mlp(
        partial[0, :N_NODES], partial[1, :N_NODES], v,
        W1.T, b1.reshape(1, HID), W2.T, b2.reshape(1, HID),
    )


# speedup vs baseline: 7.3765x; 1.1233x over previous
import functools

import jax
import jax.numpy as jnp
from jax import lax
from jax.experimental import pallas as pl
from jax.experimental.pallas import tpu as pltpu
from jax.experimental.pallas import tpu_sc as plsc

N_NODES = 10000
N_EDGES = 320000
HID = 128

NC = 2
NS = 16
NW = NC * NS
GROUP = 80
G_PER_W = N_EDGES // (NW * GROUP)
N_PAD = 10240
ROWS_PER_SUB = N_PAD // NS
ZROWS = 80


def _sc_segment_sum(e, idx3d):
    mesh = plsc.VectorSubcoreMesh(core_axis_name="c", subcore_axis_name="s")

    @functools.partial(
        pl.kernel,
        mesh=mesh,
        out_type=jax.ShapeDtypeStruct((NC, N_PAD, HID), jnp.float32),
        scratch_types=[
            pltpu.VMEM((G_PER_W, GROUP), jnp.int32),
            pltpu.VMEM((GROUP, HID), jnp.float32),
            pltpu.VMEM((GROUP, HID), jnp.float32),
            pltpu.VMEM((GROUP, HID), jnp.float32),
            pltpu.VMEM_SHARED((N_PAD, HID), jnp.float32),
            pltpu.SemaphoreType.DMA,
            pltpu.SemaphoreType.DMA,
            pltpu.SemaphoreType.DMA,
            pltpu.SemaphoreType.DMA,
            pltpu.SemaphoreType.DMA,
            pltpu.SemaphoreType.DMA,
        ],
    )
    def k(e_hbm, idx_hbm, out_hbm, idx_v, rows0_v, rows1_v, rows2_v,
          acc, sl0, sl1, sl2, ss0, ss1, ss2):
        c = lax.axis_index("c")
        s = lax.axis_index("s")
        wid = c * NS + s

        def zbody(t, carry):
            r = t // (HID // 16)
            col = t % (HID // 16)
            rows0_v[r, pl.ds(col * 16, 16)] = jnp.zeros((16,), jnp.float32)
            return carry

        lax.fori_loop(0, ZROWS * (HID // 16), zbody, 0)

        base_row = s * ROWS_PER_SUB
        for t in range(ROWS_PER_SUB // ZROWS):
            pltpu.sync_copy(rows0_v,
                            acc.at[pl.ds(base_row + t * ZROWS, ZROWS)])
        plsc.subcore_barrier()

        pltpu.sync_copy(idx_hbm.at[wid], idx_v)

        ebase = wid * G_PER_W * GROUP
        bufs = [rows0_v, rows1_v, rows2_v]
        sls = [sl0, sl1, sl2]
        sss = [ss0, ss1, ss2]

        def fire_load(g, b):
            pltpu.async_copy(e_hbm.at[pl.ds(ebase + g * GROUP, GROUP)],
                             bufs[b], sls[b])

        def drain_load(b):
            pltpu.make_async_copy(
                e_hbm.at[pl.ds(0, GROUP)], bufs[b], sls[b]).wait()

        def fire_scatter(g, b):
            pltpu.async_copy(bufs[b], acc.at[idx_v.at[g]], sss[b], add=True)

        def drain_scatter(b):
            pltpu.make_async_copy(
                bufs[b], acc.at[idx_v.at[0]], sss[b]).wait()

        fire_load(0, 0)
        fire_load(1, 1)
        drain_load(0)
        fire_scatter(0, 0)
        fire_load(2, 2)

        def body(kk, carry):
            for j in range(3):
                g = 1 + 3 * kk + j
                b = (1 + j) % 3
                drain_load(b)
                fire_scatter(g, b)
                drain_scatter((b + 2) % 3)
                fire_load(g + 2, (b + 2) % 3)
            return carry

        lax.fori_loop(0, (G_PER_W - 5) // 3, body, 0)

        drain_load(1)
        fire_scatter(G_PER_W - 4, 1)
        drain_scatter(0)
        fire_load(G_PER_W - 2, 0)
        drain_load(2)
        fire_scatter(G_PER_W - 3, 2)
        drain_scatter(1)
        fire_load(G_PER_W - 1, 1)
        drain_load(0)
        fire_scatter(G_PER_W - 2, 0)
        drain_scatter(2)
        drain_load(1)
        fire_scatter(G_PER_W - 1, 1)
        drain_scatter(0)
        drain_scatter(1)
        plsc.subcore_barrier()

        pltpu.sync_copy(
            acc.at[pl.ds(base_row, ROWS_PER_SUB)],
            out_hbm.at[c, pl.ds(base_row, ROWS_PER_SUB)],
        )

    return k(e, idx3d)


_LOG2 = 0.6931471805599453
_BLK = 1000


def _mlp_body(p0_ref, p1_ref, v_ref, w1t_ref, b1_ref, w2t_ref, b2_ref, o_ref):
    x = p0_ref[...] + p1_ref[...]
    h = jnp.dot(x, w1t_ref[...], preferred_element_type=jnp.float32) + b1_ref[...]
    h = jnp.maximum(h, 0.0) + jnp.log1p(jnp.exp(-jnp.abs(h))) - _LOG2
    y = jnp.dot(h, w2t_ref[...], preferred_element_type=jnp.float32) + b2_ref[...]
    o_ref[...] = v_ref[...] + y


def _tc_mlp(p0, p1, v, w1t, b1, w2t, b2):
    n = v.shape[0]
    grid = (n // _BLK,)
    row_spec = pl.BlockSpec((_BLK, HID), lambda i: (i, 0))
    full_spec = pl.BlockSpec((HID, HID), lambda i: (0, 0))
    bias_spec = pl.BlockSpec((1, HID), lambda i: (0, 0))
    return pl.pallas_call(
        _mlp_body,
        grid=grid,
        in_specs=[row_spec, row_spec, row_spec, full_spec, bias_spec,
                  full_spec, bias_spec],
        out_specs=row_spec,
        out_shape=jax.ShapeDtypeStruct((n, HID), jnp.float32),
    )(p0, p1, v, w1t, b1, w2t, b2)


def kernel(v, e, edge_index, W1, b1, W2, b2):
    idx3d = edge_index[1].reshape(NW, G_PER_W, GROUP)
    partial = _sc_segment_sum(e, idx3d)
    return _tc_mlp(
        partial[0, :N_NODES], partial[1, :N_NODES], v,
        W1.T, b1.reshape(1, HID), W2.T, b2.reshape(1, HID),
    )


# speedup vs baseline: 7.7364x; 1.0488x over previous
import functools

import jax
import jax.numpy as jnp
from jax import lax
from jax.experimental import pallas as pl
from jax.experimental.pallas import tpu as pltpu
from jax.experimental.pallas import tpu_sc as plsc

N_NODES = 10000
N_EDGES = 320000
HID = 128

NC = 2
NS = 16
NW = NC * NS
GROUP = 80
G_PER_W = N_EDGES // (NW * GROUP)
N_PAD = 10240
ROWS_PER_SUB = N_PAD // NS
ZROWS = 80


def _sc_segment_sum(e, idx3d):
    mesh = plsc.VectorSubcoreMesh(core_axis_name="c", subcore_axis_name="s")

    @functools.partial(
        pl.kernel,
        mesh=mesh,
        out_type=jax.ShapeDtypeStruct((NC, N_PAD, HID), jnp.float32),
        scratch_types=[
            pltpu.VMEM((G_PER_W, GROUP), jnp.int32),
            pltpu.VMEM((GROUP, HID), jnp.float32),
            pltpu.VMEM((GROUP, HID), jnp.float32),
            pltpu.VMEM((GROUP, HID), jnp.float32),
            pltpu.VMEM_SHARED((N_PAD, HID), jnp.float32),
            pltpu.SemaphoreType.DMA,
            pltpu.SemaphoreType.DMA,
            pltpu.SemaphoreType.DMA,
            pltpu.SemaphoreType.DMA,
            pltpu.SemaphoreType.DMA,
            pltpu.SemaphoreType.DMA,
        ],
    )
    def k(e_hbm, idx_hbm, out_hbm, idx_v, rows0_v, rows1_v, rows2_v,
          acc, sl0, sl1, sl2, ss0, ss1, ss2):
        c = lax.axis_index("c")
        s = lax.axis_index("s")
        wid = c * NS + s

        def zbody(t, carry):
            r = t // (HID // 16)
            col = t % (HID // 16)
            rows0_v[r, pl.ds(col * 16, 16)] = jnp.zeros((16,), jnp.float32)
            return carry

        lax.fori_loop(0, ZROWS * (HID // 16), zbody, 0)

        base_row = s * ROWS_PER_SUB
        for t in range(ROWS_PER_SUB // ZROWS):
            pltpu.sync_copy(rows0_v,
                            acc.at[pl.ds(base_row + t * ZROWS, ZROWS)])
        plsc.subcore_barrier()

        pltpu.sync_copy(idx_hbm.at[wid], idx_v)

        ebase = wid * G_PER_W * GROUP
        bufs = [rows0_v, rows1_v, rows2_v]
        sls = [sl0, sl1, sl2]
        sss = [ss0, ss1, ss2]

        def fire_load(g, b):
            pltpu.async_copy(e_hbm.at[pl.ds(ebase + g * GROUP, GROUP)],
                             bufs[b], sls[b])

        def drain_load(b):
            pltpu.make_async_copy(
                e_hbm.at[pl.ds(0, GROUP)], bufs[b], sls[b]).wait()

        def fire_scatter(g, b):
            pltpu.async_copy(bufs[b], acc.at[idx_v.at[g]], sss[b], add=True)

        def drain_scatter(b):
            pltpu.make_async_copy(
                bufs[b], acc.at[idx_v.at[0]], sss[b]).wait()

        fire_load(0, 0)
        fire_load(1, 1)
        drain_load(0)
        fire_scatter(0, 0)
        fire_load(2, 2)

        def body(kk, carry):
            for j in range(3):
                g = 1 + 3 * kk + j
                b = (1 + j) % 3
                drain_load(b)
                fire_scatter(g, b)
                drain_scatter((b + 2) % 3)
                fire_load(g + 2, (b + 2) % 3)
            return carry

        lax.fori_loop(0, (G_PER_W - 5) // 3, body, 0)

        drain_load(1)
        fire_scatter(G_PER_W - 4, 1)
        drain_scatter(0)
        fire_load(G_PER_W - 2, 0)
        drain_load(2)
        fire_scatter(G_PER_W - 3, 2)
        drain_scatter(1)
        fire_load(G_PER_W - 1, 1)
        drain_load(0)
        fire_scatter(G_PER_W - 2, 0)
        drain_scatter(2)
        drain_load(1)
        fire_scatter(G_PER_W - 1, 1)
        drain_scatter(0)
        drain_scatter(1)
        plsc.subcore_barrier()

        pltpu.sync_copy(
            acc.at[pl.ds(base_row, ROWS_PER_SUB)],
            out_hbm.at[c, pl.ds(base_row, ROWS_PER_SUB)],
        )

    return k(e, idx3d)


_LOG2 = 0.6931471805599453
_BLK = 1000


def _mlp_body(p_ref, v_ref, w1_ref, b1_ref, w2_ref, b2_ref, o_ref):
    x = p_ref[0] + p_ref[1]
    dn = (((1,), (1,)), ((), ()))
    h = lax.dot_general(x, w1_ref[...], dn,
                        preferred_element_type=jnp.float32) + b1_ref[...]
    h = jnp.maximum(h, 0.0) + jnp.log1p(jnp.exp(-jnp.abs(h))) - _LOG2
    y = lax.dot_general(h, w2_ref[...], dn,
                        preferred_element_type=jnp.float32) + b2_ref[...]
    o_ref[...] = v_ref[...] + y


def _tc_mlp(partial, v, w1, b1, w2, b2):
    n = v.shape[0]
    grid = (n // _BLK,)
    p_spec = pl.BlockSpec((NC, _BLK, HID), lambda i: (0, i, 0))
    row_spec = pl.BlockSpec((_BLK, HID), lambda i: (i, 0))
    full_spec = pl.BlockSpec((HID, HID), lambda i: (0, 0))
    bias_spec = pl.BlockSpec((1, HID), lambda i: (0, 0))
    return pl.pallas_call(
        _mlp_body,
        grid=grid,
        in_specs=[p_spec, row_spec, full_spec, bias_spec,
                  full_spec, bias_spec],
        out_specs=row_spec,
        out_shape=jax.ShapeDtypeStruct((n, HID), jnp.float32),
    )(partial, v, w1, b1, w2, b2)


def kernel(v, e, edge_index, W1, b1, W2, b2):
    idx3d = edge_index[1].reshape(NW, G_PER_W, GROUP)
    partial = _sc_segment_sum(e, idx3d)
    return _tc_mlp(partial, v, W1, b1.reshape(1, HID), W2, b2.reshape(1, HID))


# speedup vs baseline: 7.8522x; 1.0150x over previous
import functools

import jax
import jax.numpy as jnp
from jax import lax
from jax.experimental import pallas as pl
from jax.experimental.pallas import tpu as pltpu
from jax.experimental.pallas import tpu_sc as plsc

N_NODES = 10000
N_EDGES = 320000
HID = 128

NC = 2
NS = 16
NW = NC * NS
GROUP = 80
G_PER_W = N_EDGES // (NW * GROUP)
N_PAD = 10240
ROWS_PER_SUB = N_PAD // NS
ZROWS = 80


def _sc_segment_sum(e, idx3d):
    mesh = plsc.VectorSubcoreMesh(core_axis_name="c", subcore_axis_name="s")

    @functools.partial(
        pl.kernel,
        mesh=mesh,
        out_type=jax.ShapeDtypeStruct((NC, N_PAD, HID), jnp.float32),
        scratch_types=[
            pltpu.VMEM((G_PER_W, GROUP), jnp.int32),
            pltpu.VMEM((GROUP, HID), jnp.float32),
            pltpu.VMEM((GROUP, HID), jnp.float32),
            pltpu.VMEM((GROUP, HID), jnp.float32),
            pltpu.VMEM_SHARED((N_PAD, HID), jnp.float32),
            pltpu.SemaphoreType.DMA,
            pltpu.SemaphoreType.DMA,
            pltpu.SemaphoreType.DMA,
            pltpu.SemaphoreType.DMA,
            pltpu.SemaphoreType.DMA,
            pltpu.SemaphoreType.DMA,
            pltpu.SemaphoreType.DMA,
        ],
    )
    def k(e_hbm, idx_hbm, out_hbm, idx_v, rows0_v, rows1_v, rows2_v,
          acc, sl0, sl1, sl2, ss0, ss1, ss2, si):
        c = lax.axis_index("c")
        s = lax.axis_index("s")
        wid = c * NS + s
        ebase = wid * G_PER_W * GROUP

        idx_cp = pltpu.async_copy(idx_hbm.at[wid], idx_v, si)
        pltpu.async_copy(e_hbm.at[pl.ds(ebase + 1 * GROUP, GROUP)],
                         rows1_v, sl1)
        pltpu.async_copy(e_hbm.at[pl.ds(ebase + 2 * GROUP, GROUP)],
                         rows2_v, sl2)

        def zrow(r, carry):
            for c16 in range(HID // 16):
                rows0_v[r, pl.ds(c16 * 16, 16)] = jnp.zeros((16,), jnp.float32)
            return carry

        lax.fori_loop(0, ZROWS, zrow, 0)

        base_row = s * ROWS_PER_SUB
        for t in range(ROWS_PER_SUB // ZROWS):
            pltpu.async_copy(rows0_v,
                             acc.at[pl.ds(base_row + t * ZROWS, ZROWS)], ss0)
        for t in range(ROWS_PER_SUB // ZROWS):
            pltpu.make_async_copy(rows0_v, acc.at[pl.ds(0, ZROWS)],
                                  ss0).wait()

        pltpu.async_copy(e_hbm.at[pl.ds(ebase, GROUP)], rows0_v, sl0)
        idx_cp.wait()
        plsc.subcore_barrier()

        bufs = [rows0_v, rows1_v, rows2_v]
        sls = [sl0, sl1, sl2]
        sss = [ss0, ss1, ss2]

        def fire_load(g, b):
            pltpu.async_copy(e_hbm.at[pl.ds(ebase + g * GROUP, GROUP)],
                             bufs[b], sls[b])

        def drain_load(b):
            pltpu.make_async_copy(
                e_hbm.at[pl.ds(0, GROUP)], bufs[b], sls[b]).wait()

        def fire_scatter(g, b):
            pltpu.async_copy(bufs[b], acc.at[idx_v.at[g]], sss[b], add=True)

        def drain_scatter(b):
            pltpu.make_async_copy(
                bufs[b], acc.at[idx_v.at[0]], sss[b]).wait()

        drain_load(0)
        fire_scatter(0, 0)

        def body(kk, carry):
            for j in range(12):
                g = 1 + 12 * kk + j
                b = (1 + j) % 3
                drain_load(b)
                fire_scatter(g, b)
                drain_scatter((b + 2) % 3)
                fire_load(g + 2, (b + 2) % 3)
            return carry

        lax.fori_loop(0, (G_PER_W - 5) // 12, body, 0)

        drain_load(1)
        fire_scatter(G_PER_W - 4, 1)
        drain_scatter(0)
        fire_load(G_PER_W - 2, 0)
        drain_load(2)
        fire_scatter(G_PER_W - 3, 2)
        drain_scatter(1)
        fire_load(G_PER_W - 1, 1)
        drain_load(0)
        fire_scatter(G_PER_W - 2, 0)
        drain_scatter(2)
        drain_load(1)
        fire_scatter(G_PER_W - 1, 1)
        drain_scatter(0)
        drain_scatter(1)
        plsc.subcore_barrier()

        pltpu.sync_copy(
            acc.at[pl.ds(base_row, ROWS_PER_SUB)],
            out_hbm.at[c, pl.ds(base_row, ROWS_PER_SUB)],
        )

    return k(e, idx3d)


_LOG2 = 0.6931471805599453
_BLK = 1000


def _mlp_body(p_ref, v_ref, w1_ref, b1_ref, w2_ref, b2_ref, o_ref):
    x = p_ref[0] + p_ref[1]
    dn = (((1,), (1,)), ((), ()))
    h = lax.dot_general(x, w1_ref[...], dn,
                        preferred_element_type=jnp.float32) + b1_ref[...]
    h = jnp.maximum(h, 0.0) + jnp.log1p(jnp.exp(-jnp.abs(h))) - _LOG2
    y = lax.dot_general(h, w2_ref[...], dn,
                        preferred_element_type=jnp.float32) + b2_ref[...]
    o_ref[...] = v_ref[...] + y


def _tc_mlp(partial, v, w1, b1, w2, b2):
    n = v.shape[0]
    grid = (n // _BLK,)
    p_spec = pl.BlockSpec((NC, _BLK, HID), lambda i: (0, i, 0))
    row_spec = pl.BlockSpec((_BLK, HID), lambda i: (i, 0))
    full_spec = pl.BlockSpec((HID, HID), lambda i: (0, 0))
    bias_spec = pl.BlockSpec((1, HID), lambda i: (0, 0))
    return pl.pallas_call(
        _mlp_body,
        grid=grid,
        in_specs=[p_spec, row_spec, full_spec, bias_spec,
                  full_spec, bias_spec],
        out_specs=row_spec,
        out_shape=jax.ShapeDtypeStruct((n, HID), jnp.float32),
    )(partial, v, w1, b1, w2, b2)


def kernel(v, e, edge_index, W1, b1, W2, b2):
    idx3d = edge_index[1].reshape(NW, G_PER_W, GROUP)
    partial = _sc_segment_sum(e, idx3d)
    return _tc_mlp(partial, v, W1, b1.reshape(1, HID), W2, b2.reshape(1, HID))
